# baseline (speedup 1.0000x reference)
import functools

import jax
import jax.numpy as jnp
import numpy as np
from jax.experimental import pallas as pl
from jax.experimental.pallas import tpu as pltpu

NUM_CLASSES = 6
BN_EPS = 1e-5
BN_SCALE = 1.0 / np.sqrt(1.0 + BN_EPS)


def _square_distance(src, dst):
    return (
        jnp.sum(src ** 2, -1)[:, :, None]
        + jnp.sum(dst ** 2, -1)[:, None, :]
        - 2.0 * jnp.einsum('bnc,bmc->bnm', src, dst)
    )


def _index_points(points, idx):
    return jax.vmap(lambda p, i: p[i])(points, idx)


def _farthest_point_sample(xyz, npoint):
    B, N, _ = xyz.shape

    def step(state, _):
        distance, farthest = state
        centroid = _index_points(xyz, farthest[:, None])
        dist = jnp.sum((xyz - centroid) ** 2, -1)
        distance = jnp.minimum(distance, dist)
        new_far = jnp.argmax(distance, axis=-1).astype(jnp.int32)
        return (distance, new_far), farthest

    init = (jnp.full((B, N), 1e10, xyz.dtype), jnp.zeros((B,), jnp.int32))
    _, cent = jax.lax.scan(step, init, None, length=npoint)
    return jnp.transpose(cent)


def _query_ball_point(radius, nsample, xyz, new_xyz):
    B, N, _ = xyz.shape
    S = new_xyz.shape[1]
    sqrdists = _square_distance(new_xyz, xyz)
    gidx = jnp.broadcast_to(jnp.arange(N, dtype=jnp.int32), (B, S, N))
    gidx = jnp.where(sqrdists > radius ** 2, N, gidx)
    gidx = jnp.sort(gidx, axis=-1)[:, :, :nsample]
    first = jnp.broadcast_to(gidx[:, :, :1], gidx.shape)
    return jnp.where(gidx == N, first, gidx)


def _mlp2d(x, layers):
    for p in layers:
        x = jnp.einsum('oc,bcns->bons', p['W'], x) + p['b'][None, :, None, None]
        x = x * BN_SCALE * p['gamma'][None, :, None, None] + p['beta'][None, :, None, None]
        x = jax.nn.relu(x)
    return x


def _mlp1d(x, layers):
    for p in layers:
        x = jnp.einsum('oc,bcn->bon', p['W'], x) + p['b'][None, :, None]
        x = x * BN_SCALE * p['gamma'][None, :, None] + p['beta'][None, :, None]
        x = jax.nn.relu(x)
    return x


def _sa_msg(xyz, points, npoint, radii, nsamples, branches):
    xyz_t = jnp.transpose(xyz, (0, 2, 1))
    pts_t = jnp.transpose(points, (0, 2, 1))
    fps_idx = _farthest_point_sample(xyz_t, npoint)
    new_xyz = _index_points(xyz_t, fps_idx)
    outs = []
    for radius, nsample, layers in zip(radii, nsamples, branches):
        gidx = _query_ball_point(radius, nsample, xyz_t, new_xyz)
        grouped_xyz = _index_points(xyz_t, gidx) - new_xyz[:, :, None, :]
        grouped = jnp.concatenate([_index_points(pts_t, gidx), grouped_xyz], axis=-1)
        h = jnp.transpose(grouped, (0, 3, 2, 1))
        h = _mlp2d(h, layers)
        outs.append(jnp.max(h, axis=2))
    return jnp.transpose(new_xyz, (0, 2, 1)), jnp.concatenate(outs, axis=1)


def _sa_group_all(xyz, points, layers):
    xyz_t = jnp.transpose(xyz, (0, 2, 1))
    pts_t = jnp.transpose(points, (0, 2, 1))
    B = xyz_t.shape[0]
    grouped = jnp.concatenate([xyz_t, pts_t], axis=-1)[:, None, :, :]
    h = jnp.transpose(grouped, (0, 3, 2, 1))
    h = _mlp2d(h, layers)
    new_points = jnp.max(h, axis=2)
    return jnp.zeros((B, 3, 1), xyz.dtype), new_points


def _feature_propagation(xyz1, xyz2, points1, points2, layers):
    xyz1_t = jnp.transpose(xyz1, (0, 2, 1))
    xyz2_t = jnp.transpose(xyz2, (0, 2, 1))
    pts2_t = jnp.transpose(points2, (0, 2, 1))
    N = xyz1_t.shape[1]
    S = xyz2_t.shape[1]
    if S == 1:
        interpolated = jnp.repeat(pts2_t, N, axis=1)
    else:
        dists = _square_distance(xyz1_t, xyz2_t)
        idx = jnp.argsort(dists, axis=-1)[:, :, :3]
        d = jnp.take_along_axis(dists, idx, axis=-1)
        recip = 1.0 / (d + 1e-8)
        weight = recip / jnp.sum(recip, axis=2, keepdims=True)
        interpolated = jnp.sum(_index_points(pts2_t, idx) * weight[..., None], axis=2)
    new_points = jnp.concatenate([jnp.transpose(points1, (0, 2, 1)), interpolated], axis=-1)
    return _mlp1d(jnp.transpose(new_points, (0, 2, 1)), layers)


def _head_body(x_ref, w1_ref, b1_ref, w2_ref, b2_ref, o_ref):
    x = x_ref[0]
    h = jnp.dot(w1_ref[...], x, preferred_element_type=jnp.float32)
    h = h + b1_ref[...][:, :1]
    h = jax.nn.relu(h)
    h = jnp.dot(w2_ref[...], h, preferred_element_type=jnp.float32)
    h = h + b2_ref[...][:, :1]
    o_ref[0] = jax.nn.sigmoid(h)


def _head_pallas(l0p, head1, head2):
    B, C, N = l0p.shape
    BLK = 1024
    w1 = head1['W'] * (BN_SCALE * head1['gamma'])[:, None]
    b1 = (head1['b'] * BN_SCALE * head1['gamma'] + head1['beta'])[:, None]
    w2 = head2['W']
    b2 = head2['b'][:, None]
    out = pl.pallas_call(
        _head_body,
        grid=(B, N // BLK),
        in_specs=[
            pl.BlockSpec((1, C, BLK), lambda b, n: (b, 0, n)),
            pl.BlockSpec((w1.shape[0], C), lambda b, n: (0, 0)),
            pl.BlockSpec((w1.shape[0], 1), lambda b, n: (0, 0)),
            pl.BlockSpec((8, w2.shape[1]), lambda b, n: (0, 0)),
            pl.BlockSpec((8, 1), lambda b, n: (0, 0)),
        ],
        out_specs=pl.BlockSpec((1, 8, BLK), lambda b, n: (b, 0, n)),
        out_shape=jax.ShapeDtypeStruct((B, 8, N), jnp.float32),
    )(l0p, w1, b1, jnp.pad(w2, ((0, 2), (0, 0))), jnp.pad(b2, ((0, 2), (0, 0))))
    return out[:, :NUM_CLASSES, :]


def kernel(point_cloud_xyz, params):
    input_points = jnp.transpose(point_cloud_xyz, (0, 2, 1))
    input_xyz = input_points
    l1_xyz, l1_points = _sa_msg(input_xyz, input_points, 512, [0.1, 0.2, 0.4], [32, 64, 128], params['sa1'])
    l2_xyz, l2_points = _sa_msg(l1_xyz, l1_points, 128, [0.4, 0.8], [64, 128], params['sa2'])
    l3_xyz, l3_points = _sa_group_all(l2_xyz, l2_points, params['sa3'])
    l2p = _feature_propagation(l2_xyz, l3_xyz, l2_points, l3_points, params['fp3'])
    l1p = _feature_propagation(l1_xyz, l2_xyz, l1_points, l2p, params['fp2'])
    l0p = _feature_propagation(input_xyz, l1_xyz, input_points, l1p, params['fp1'])
    h = _head_pallas(l0p, params['head1'], params['head2'])
    return jnp.transpose(h, (0, 2, 1)), l3_points


# speedup vs baseline: 1.1771x; 1.1771x over previous
import functools

import jax
import jax.numpy as jnp
import numpy as np
from jax.experimental import pallas as pl
from jax.experimental.pallas import tpu as pltpu

NUM_CLASSES = 6
BN_EPS = 1e-5
BN_SCALE = 1.0 / np.sqrt(1.0 + BN_EPS)


def _square_distance(src, dst):
    return (
        jnp.sum(src ** 2, -1)[:, :, None]
        + jnp.sum(dst ** 2, -1)[:, None, :]
        - 2.0 * jnp.einsum('bnc,bmc->bnm', src, dst)
    )


def _index_points(points, idx):
    return jax.vmap(lambda p, i: p[i])(points, idx)


def _fps_body(npoint, x_ref, o_ref):
    B = x_ref.shape[0] // 3
    N = x_ref.shape[1]
    x = x_ref[0:B, :]
    y = x_ref[B:2 * B, :]
    z = x_ref[2 * B:3 * B, :]
    lane = jax.lax.broadcasted_iota(jnp.int32, (B, N), 1).astype(jnp.float32)
    lane_s = jax.lax.broadcasted_iota(jnp.int32, (B, npoint), 1)

    def step(s, state):
        distance, far, ax, ay, az = state
        oh = (lane == far).astype(jnp.float32)
        cx = jnp.sum(x * oh, axis=1, keepdims=True)
        cy = jnp.sum(y * oh, axis=1, keepdims=True)
        cz = jnp.sum(z * oh, axis=1, keepdims=True)
        ohs = (lane_s == s).astype(jnp.float32)
        ax = ax + cx * ohs
        ay = ay + cy * ohs
        az = az + cz * ohs
        dx = x - cx
        dy = y - cy
        dz = z - cz
        dist = dx * dx + dy * dy + dz * dz
        distance = jnp.minimum(distance, dist)
        m = jnp.max(distance, axis=1, keepdims=True)
        far = jnp.min(jnp.where(distance == m, lane, float(N)), axis=1, keepdims=True)
        return distance, far, ax, ay, az

    init = (
        jnp.full((B, N), 1e10, jnp.float32),
        jnp.zeros((B, 1), jnp.float32),
        jnp.zeros((B, npoint), jnp.float32),
        jnp.zeros((B, npoint), jnp.float32),
        jnp.zeros((B, npoint), jnp.float32),
    )
    _, _, ax, ay, az = jax.lax.fori_loop(0, npoint, step, init)
    o_ref[0:B, :] = ax
    o_ref[B:2 * B, :] = ay
    o_ref[2 * B:3 * B, :] = az


def _fps_new_xyz(xyz_t, npoint):
    B, N, _ = xyz_t.shape
    planes = jnp.transpose(xyz_t, (2, 0, 1)).reshape(3 * B, N)
    out = pl.pallas_call(
        functools.partial(_fps_body, npoint),
        in_specs=[pl.BlockSpec((3 * B, N), lambda: (0, 0))],
        out_specs=pl.BlockSpec((3 * B, npoint), lambda: (0, 0)),
        out_shape=jax.ShapeDtypeStruct((3 * B, npoint), jnp.float32),
    )(planes)
    return jnp.transpose(out.reshape(3, B, npoint), (1, 2, 0))


def _query_ball_point(radius, nsample, xyz, new_xyz):
    B, N, _ = xyz.shape
    S = new_xyz.shape[1]
    sqrdists = _square_distance(new_xyz, xyz)
    gidx = jnp.broadcast_to(jnp.arange(N, dtype=jnp.int32), (B, S, N))
    gidx = jnp.where(sqrdists > radius ** 2, N, gidx)
    gidx = jnp.sort(gidx, axis=-1)[:, :, :nsample]
    first = jnp.broadcast_to(gidx[:, :, :1], gidx.shape)
    return jnp.where(gidx == N, first, gidx)


def _mlp2d(x, layers):
    for p in layers:
        x = jnp.einsum('oc,bcns->bons', p['W'], x) + p['b'][None, :, None, None]
        x = x * BN_SCALE * p['gamma'][None, :, None, None] + p['beta'][None, :, None, None]
        x = jax.nn.relu(x)
    return x


def _mlp1d(x, layers):
    for p in layers:
        x = jnp.einsum('oc,bcn->bon', p['W'], x) + p['b'][None, :, None]
        x = x * BN_SCALE * p['gamma'][None, :, None] + p['beta'][None, :, None]
        x = jax.nn.relu(x)
    return x


def _sa_msg(xyz, points, npoint, radii, nsamples, branches):
    xyz_t = jnp.transpose(xyz, (0, 2, 1))
    pts_t = jnp.transpose(points, (0, 2, 1))
    new_xyz = _fps_new_xyz(xyz_t, npoint)
    outs = []
    for radius, nsample, layers in zip(radii, nsamples, branches):
        gidx = _query_ball_point(radius, nsample, xyz_t, new_xyz)
        grouped_xyz = _index_points(xyz_t, gidx) - new_xyz[:, :, None, :]
        grouped = jnp.concatenate([_index_points(pts_t, gidx), grouped_xyz], axis=-1)
        h = jnp.transpose(grouped, (0, 3, 2, 1))
        h = _mlp2d(h, layers)
        outs.append(jnp.max(h, axis=2))
    return jnp.transpose(new_xyz, (0, 2, 1)), jnp.concatenate(outs, axis=1)


def _sa_group_all(xyz, points, layers):
    xyz_t = jnp.transpose(xyz, (0, 2, 1))
    pts_t = jnp.transpose(points, (0, 2, 1))
    B = xyz_t.shape[0]
    grouped = jnp.concatenate([xyz_t, pts_t], axis=-1)[:, None, :, :]
    h = jnp.transpose(grouped, (0, 3, 2, 1))
    h = _mlp2d(h, layers)
    new_points = jnp.max(h, axis=2)
    return jnp.zeros((B, 3, 1), xyz.dtype), new_points


def _feature_propagation(xyz1, xyz2, points1, points2, layers):
    xyz1_t = jnp.transpose(xyz1, (0, 2, 1))
    xyz2_t = jnp.transpose(xyz2, (0, 2, 1))
    pts2_t = jnp.transpose(points2, (0, 2, 1))
    N = xyz1_t.shape[1]
    S = xyz2_t.shape[1]
    if S == 1:
        interpolated = jnp.repeat(pts2_t, N, axis=1)
    else:
        dists = _square_distance(xyz1_t, xyz2_t)
        idx = jnp.argsort(dists, axis=-1)[:, :, :3]
        d = jnp.take_along_axis(dists, idx, axis=-1)
        recip = 1.0 / (d + 1e-8)
        weight = recip / jnp.sum(recip, axis=2, keepdims=True)
        interpolated = jnp.sum(_index_points(pts2_t, idx) * weight[..., None], axis=2)
    new_points = jnp.concatenate([jnp.transpose(points1, (0, 2, 1)), interpolated], axis=-1)
    return _mlp1d(jnp.transpose(new_points, (0, 2, 1)), layers)


def _head_body(x_ref, w1_ref, b1_ref, w2_ref, b2_ref, o_ref):
    x = x_ref[0]
    h = jnp.dot(w1_ref[...], x, preferred_element_type=jnp.float32)
    h = h + b1_ref[...][:, :1]
    h = jax.nn.relu(h)
    h = jnp.dot(w2_ref[...], h, preferred_element_type=jnp.float32)
    h = h + b2_ref[...][:, :1]
    o_ref[0] = jax.nn.sigmoid(h)


def _head_pallas(l0p, head1, head2):
    B, C, N = l0p.shape
    BLK = 1024
    w1 = head1['W'] * (BN_SCALE * head1['gamma'])[:, None]
    b1 = (head1['b'] * BN_SCALE * head1['gamma'] + head1['beta'])[:, None]
    w2 = head2['W']
    b2 = head2['b'][:, None]
    out = pl.pallas_call(
        _head_body,
        grid=(B, N // BLK),
        in_specs=[
            pl.BlockSpec((1, C, BLK), lambda b, n: (b, 0, n)),
            pl.BlockSpec((w1.shape[0], C), lambda b, n: (0, 0)),
            pl.BlockSpec((w1.shape[0], 1), lambda b, n: (0, 0)),
            pl.BlockSpec((8, w2.shape[1]), lambda b, n: (0, 0)),
            pl.BlockSpec((8, 1), lambda b, n: (0, 0)),
        ],
        out_specs=pl.BlockSpec((1, 8, BLK), lambda b, n: (b, 0, n)),
        out_shape=jax.ShapeDtypeStruct((B, 8, N), jnp.float32),
    )(l0p, w1, b1, jnp.pad(w2, ((0, 2), (0, 0))), jnp.pad(b2, ((0, 2), (0, 0))))
    return out[:, :NUM_CLASSES, :]


def kernel(point_cloud_xyz, params):
    input_points = jnp.transpose(point_cloud_xyz, (0, 2, 1))
    input_xyz = input_points
    l1_xyz, l1_points = _sa_msg(input_xyz, input_points, 512, [0.1, 0.2, 0.4], [32, 64, 128], params['sa1'])
    l2_xyz, l2_points = _sa_msg(l1_xyz, l1_points, 128, [0.4, 0.8], [64, 128], params['sa2'])
    l3_xyz, l3_points = _sa_group_all(l2_xyz, l2_points, params['sa3'])
    l2p = _feature_propagation(l2_xyz, l3_xyz, l2_points, l3_points, params['fp3'])
    l1p = _feature_propagation(l1_xyz, l2_xyz, l1_points, l2p, params['fp2'])
    l0p = _feature_propagation(input_xyz, l1_xyz, input_points, l1p, params['fp1'])
    h = _head_pallas(l0p, params['head1'], params['head2'])
    return jnp.transpose(h, (0, 2, 1)), l3_points


# speedup vs baseline: 2.6966x; 2.2909x over previous
import functools

import jax
import jax.numpy as jnp
import numpy as np
from jax.experimental import pallas as pl
from jax.experimental.pallas import tpu as pltpu

NUM_CLASSES = 6
BN_EPS = 1e-5
BN_SCALE = 1.0 / np.sqrt(1.0 + BN_EPS)


def _square_distance(src, dst):
    return (
        jnp.sum(src ** 2, -1)[:, :, None]
        + jnp.sum(dst ** 2, -1)[:, None, :]
        - 2.0 * jnp.einsum('bnc,bmc->bnm', src, dst)
    )


def _index_points(points, idx):
    return jax.vmap(lambda p, i: p[i])(points, idx)


def _fps_body(npoint, x_ref, o_ref):
    B = x_ref.shape[0] // 3
    N = x_ref.shape[1]
    x = x_ref[0:B, :]
    y = x_ref[B:2 * B, :]
    z = x_ref[2 * B:3 * B, :]
    lane = jax.lax.broadcasted_iota(jnp.int32, (B, N), 1).astype(jnp.float32)
    lane_s = jax.lax.broadcasted_iota(jnp.int32, (B, npoint), 1)

    def step(s, state):
        distance, far, ax, ay, az = state
        oh = (lane == far).astype(jnp.float32)
        cx = jnp.sum(x * oh, axis=1, keepdims=True)
        cy = jnp.sum(y * oh, axis=1, keepdims=True)
        cz = jnp.sum(z * oh, axis=1, keepdims=True)
        ohs = (lane_s == s).astype(jnp.float32)
        ax = ax + cx * ohs
        ay = ay + cy * ohs
        az = az + cz * ohs
        dx = x - cx
        dy = y - cy
        dz = z - cz
        dist = dx * dx + dy * dy + dz * dz
        distance = jnp.minimum(distance, dist)
        m = jnp.max(distance, axis=1, keepdims=True)
        far = jnp.min(jnp.where(distance == m, lane, float(N)), axis=1, keepdims=True)
        return distance, far, ax, ay, az

    init = (
        jnp.full((B, N), 1e10, jnp.float32),
        jnp.zeros((B, 1), jnp.float32),
        jnp.zeros((B, npoint), jnp.float32),
        jnp.zeros((B, npoint), jnp.float32),
        jnp.zeros((B, npoint), jnp.float32),
    )
    _, _, ax, ay, az = jax.lax.fori_loop(0, npoint, step, init)
    o_ref[0:B, :] = ax
    o_ref[B:2 * B, :] = ay
    o_ref[2 * B:3 * B, :] = az


def _fps_new_xyz(xyz_t, npoint):
    B, N, _ = xyz_t.shape
    planes = jnp.transpose(xyz_t, (2, 0, 1)).reshape(3 * B, N)
    out = pl.pallas_call(
        functools.partial(_fps_body, npoint),
        in_specs=[pl.BlockSpec((3 * B, N), lambda: (0, 0))],
        out_specs=pl.BlockSpec((3 * B, npoint), lambda: (0, 0)),
        out_shape=jax.ShapeDtypeStruct((3 * B, npoint), jnp.float32),
    )(planes)
    return jnp.transpose(out.reshape(3, B, npoint), (1, 2, 0))


def _ballq_body(radii, ksamples, q_ref, x3_ref, *o_refs):
    q = q_ref[0]
    x3 = x3_ref[0]
    SB = q.shape[0]
    N = x3.shape[1]
    C = 128
    NC = N // C
    qn = jnp.sum(q * q, axis=1, keepdims=True)
    xn = jnp.sum(x3 * x3, axis=0, keepdims=True)
    qx = jax.lax.dot_general(q, x3, (((1,), (0,)), ((), ())),
                             preferred_element_type=jnp.float32)
    sqd = qn + xn - 2.0 * qx

    ii = jax.lax.broadcasted_iota(jnp.int32, (C, C), 0)
    jj = jax.lax.broadcasted_iota(jnp.int32, (C, C), 1)
    t128 = (ii <= jj).astype(jnp.bfloat16)
    i64 = jax.lax.broadcasted_iota(jnp.int32, (NC, NC), 0)
    j64 = jax.lax.broadcasted_iota(jnp.int32, (NC, NC), 1)
    t64 = (i64 <= j64).astype(jnp.bfloat16)
    c_iota = jax.lax.broadcasted_iota(jnp.int32, (1, NC, 1), 1).astype(jnp.float32)

    for i, (radius, K) in enumerate(zip(radii, ksamples)):
        kio = jax.lax.broadcasted_iota(jnp.int32, (SB, K), 1).astype(jnp.float32)
        mask = (sqd <= radius * radius).astype(jnp.bfloat16)
        rloc = jnp.dot(mask.reshape(SB * NC, C), t128,
                       preferred_element_type=jnp.float32)
        rloc3 = rloc.reshape(SB, NC, C)
        cnt = rloc3[:, :, C - 1]
        y = jnp.dot(cnt.astype(jnp.bfloat16), t64,
                    preferred_element_type=jnp.float32)
        xp = y - cnt
        ch = jnp.sum((y[:, :, None] <= kio[:, None, :]).astype(jnp.float32),
                     axis=1)
        chc = jnp.minimum(ch, float(NC - 1))
        oh = (chc[:, None, :] == c_iota).astype(jnp.float32)
        base = jnp.sum(oh * xp[:, :, None], axis=1)
        g = jax.lax.dot_general(
            oh.astype(jnp.bfloat16), rloc3.astype(jnp.bfloat16),
            (((1,), (1,)), ((0,), (0,))),
            preferred_element_type=jnp.float32)
        t = kio - base + 1.0
        lpos = jnp.sum((g < t[:, :, None]).astype(jnp.float32), axis=2)
        idx = chc * float(C) + lpos
        total = y[:, NC - 1][:, None]
        idx = jnp.where(kio < total, idx, idx[:, 0:1])
        o_refs[i][0] = idx.astype(jnp.int32)


def _query_ball_multi(radii, ksamples, xyz_t, new_xyz):
    B, N, _ = xyz_t.shape
    S = new_xyz.shape[1]
    SB = min(S, 128)
    x3 = jnp.transpose(xyz_t, (0, 2, 1))
    outs = pl.pallas_call(
        functools.partial(_ballq_body, radii, ksamples),
        grid=(B, S // SB),
        in_specs=[
            pl.BlockSpec((1, SB, 3), lambda b, s: (b, s, 0)),
            pl.BlockSpec((1, 3, N), lambda b, s: (b, 0, 0)),
        ],
        out_specs=[pl.BlockSpec((1, SB, K), lambda b, s: (b, s, 0))
                   for K in ksamples],
        out_shape=[jax.ShapeDtypeStruct((B, S, K), jnp.int32)
                   for K in ksamples],
    )(new_xyz, x3)
    return outs


def _mlp2d(x, layers):
    for p in layers:
        x = jnp.einsum('oc,bcns->bons', p['W'], x) + p['b'][None, :, None, None]
        x = x * BN_SCALE * p['gamma'][None, :, None, None] + p['beta'][None, :, None, None]
        x = jax.nn.relu(x)
    return x


def _mlp1d(x, layers):
    for p in layers:
        x = jnp.einsum('oc,bcn->bon', p['W'], x) + p['b'][None, :, None]
        x = x * BN_SCALE * p['gamma'][None, :, None] + p['beta'][None, :, None]
        x = jax.nn.relu(x)
    return x


def _sa_msg(xyz, points, npoint, radii, nsamples, branches):
    xyz_t = jnp.transpose(xyz, (0, 2, 1))
    pts_t = jnp.transpose(points, (0, 2, 1))
    new_xyz = _fps_new_xyz(xyz_t, npoint)
    gidxs = _query_ball_multi(radii, nsamples, xyz_t, new_xyz)
    outs = []
    for gidx, nsample, layers in zip(gidxs, nsamples, branches):
        grouped_xyz = _index_points(xyz_t, gidx) - new_xyz[:, :, None, :]
        grouped = jnp.concatenate([_index_points(pts_t, gidx), grouped_xyz], axis=-1)
        h = jnp.transpose(grouped, (0, 3, 2, 1))
        h = _mlp2d(h, layers)
        outs.append(jnp.max(h, axis=2))
    return jnp.transpose(new_xyz, (0, 2, 1)), jnp.concatenate(outs, axis=1)


def _sa_group_all(xyz, points, layers):
    xyz_t = jnp.transpose(xyz, (0, 2, 1))
    pts_t = jnp.transpose(points, (0, 2, 1))
    B = xyz_t.shape[0]
    grouped = jnp.concatenate([xyz_t, pts_t], axis=-1)[:, None, :, :]
    h = jnp.transpose(grouped, (0, 3, 2, 1))
    h = _mlp2d(h, layers)
    new_points = jnp.max(h, axis=2)
    return jnp.zeros((B, 3, 1), xyz.dtype), new_points


def _feature_propagation(xyz1, xyz2, points1, points2, layers):
    xyz1_t = jnp.transpose(xyz1, (0, 2, 1))
    xyz2_t = jnp.transpose(xyz2, (0, 2, 1))
    pts2_t = jnp.transpose(points2, (0, 2, 1))
    N = xyz1_t.shape[1]
    S = xyz2_t.shape[1]
    if S == 1:
        interpolated = jnp.repeat(pts2_t, N, axis=1)
    else:
        dists = _square_distance(xyz1_t, xyz2_t)
        idx = jnp.argsort(dists, axis=-1)[:, :, :3]
        d = jnp.take_along_axis(dists, idx, axis=-1)
        recip = 1.0 / (d + 1e-8)
        weight = recip / jnp.sum(recip, axis=2, keepdims=True)
        interpolated = jnp.sum(_index_points(pts2_t, idx) * weight[..., None], axis=2)
    new_points = jnp.concatenate([jnp.transpose(points1, (0, 2, 1)), interpolated], axis=-1)
    return _mlp1d(jnp.transpose(new_points, (0, 2, 1)), layers)


def _fuse_bn(p):
    s = BN_SCALE * p['gamma']
    wt = (p['W'] * s[:, None]).T
    b = (p['b'] * s + p['beta'])[None, :]
    return wt, b


def _fp_body(nlayers, nheads, q_ref, x3_ref, p1_ref, p2_ref, *refs):
    w_refs = refs[:-1]
    o_ref = refs[-1]
    q = q_ref[0]
    x3 = x3_ref[0]
    S = x3.shape[1]
    NB = q.shape[0]
    qn = jnp.sum(q * q, axis=1, keepdims=True)
    xn = jnp.sum(x3 * x3, axis=0, keepdims=True)
    qx = jax.lax.dot_general(q, x3, (((1,), (0,)), ((), ())),
                             preferred_element_type=jnp.float32)
    sqd = qn + xn - 2.0 * qx
    lane = jax.lax.broadcasted_iota(jnp.int32, (NB, S), 1).astype(jnp.float32)
    p2 = p2_ref[0]
    interp = None
    rsum = None
    dists = sqd
    parts = []
    for _ in range(3):
        m = jnp.min(dists, axis=1, keepdims=True)
        pos = jnp.min(jnp.where(dists == m, lane, float(S)), axis=1, keepdims=True)
        oh = (lane == pos).astype(jnp.float32)
        gath = jnp.dot(oh, p2, preferred_element_type=jnp.float32)
        r = 1.0 / (m + 1e-8)
        parts.append((r, gath))
        rsum = r if rsum is None else rsum + r
        dists = jnp.where(oh > 0, jnp.float32(3.4e38), dists)
    interp = sum((r / rsum) * g for r, g in parts)
    h = jnp.concatenate([p1_ref[0], interp], axis=1)
    for li in range(nlayers):
        w, b = w_refs[2 * li][...], w_refs[2 * li + 1][...]
        h = jnp.dot(h, w, preferred_element_type=jnp.float32) + b
        h = jax.nn.relu(h)
    if nheads:
        w, b = w_refs[2 * nlayers][...], w_refs[2 * nlayers + 1][...]
        h = jax.nn.relu(jnp.dot(h, w, preferred_element_type=jnp.float32) + b)
        w, b = w_refs[2 * nlayers + 2][...], w_refs[2 * nlayers + 3][...]
        h = jax.nn.sigmoid(jnp.dot(h, w, preferred_element_type=jnp.float32) + b)
    o_ref[0] = h


def _fp_pallas(xyz1_t, xyz2_t, pts1_t, pts2_t, layers, heads=None, nblk=None):
    B, N, _ = xyz1_t.shape
    S = xyz2_t.shape[1]
    C1 = pts1_t.shape[2]
    C2 = pts2_t.shape[2]
    NB = nblk or N
    x3 = jnp.transpose(xyz2_t, (0, 2, 1))
    wbs = []
    for p in layers:
        wt, b = _fuse_bn(p)
        wbs += [wt, b]
    nheads = 0
    if heads is not None:
        h1, h2 = heads
        wt, b = _fuse_bn(h1)
        wbs += [wt, b]
        w2 = jnp.pad(h2['W'].T, ((0, 0), (0, 8 - NUM_CLASSES)))
        b2 = jnp.pad(h2['b'][None, :], ((0, 0), (0, 8 - NUM_CLASSES)))
        wbs += [w2, b2]
        nheads = 2
    cout = wbs[-2].shape[1]
    in_specs = [
        pl.BlockSpec((1, NB, 3), lambda b, n: (b, n, 0)),
        pl.BlockSpec((1, 3, S), lambda b, n: (b, 0, 0)),
        pl.BlockSpec((1, NB, C1), lambda b, n: (b, n, 0)),
        pl.BlockSpec((1, S, C2), lambda b, n: (b, 0, 0)),
    ]
    for wb in wbs:
        in_specs.append(pl.BlockSpec(wb.shape, lambda b, n: tuple([0] * wb.ndim)))
    out = pl.pallas_call(
        functools.partial(_fp_body, len(layers), nheads),
        grid=(B, N // NB),
        in_specs=in_specs,
        out_specs=pl.BlockSpec((1, NB, cout), lambda b, n: (b, n, 0)),
        out_shape=jax.ShapeDtypeStruct((B, N, cout), jnp.float32),
    )(xyz1_t, x3, pts1_t, pts2_t, *wbs)
    return out


def _sa3_fp3_body(p1_ref, px_ref, *refs):
    w_refs = refs[:-2]
    l3_ref, o_ref = refs[-2], refs[-1]
    h = p1_ref[0]
    for li in range(3):
        w, b = w_refs[2 * li][...], w_refs[2 * li + 1][...]
        h = jax.nn.relu(jnp.dot(h, w, preferred_element_type=jnp.float32) + b)
    l3 = jnp.max(h, axis=0, keepdims=True)
    l3_ref[0] = l3
    S = px_ref.shape[1]
    g = jnp.concatenate(
        [px_ref[0], jnp.broadcast_to(l3, (S, l3.shape[1]))], axis=1)
    for li in range(3, 5):
        w, b = w_refs[2 * li][...], w_refs[2 * li + 1][...]
        g = jax.nn.relu(jnp.dot(g, w, preferred_element_type=jnp.float32) + b)
    o_ref[0] = g


def _sa3_fp3_pallas(l2_xyz_t, l2_pts_t, sa3_layers, fp3_layers):
    B, S, _ = l2_xyz_t.shape
    grouped = jnp.concatenate([l2_xyz_t, l2_pts_t], axis=2)
    wbs = []
    for p in list(sa3_layers) + list(fp3_layers):
        wt, b = _fuse_bn(p)
        wbs += [wt, b]
    in_specs = [
        pl.BlockSpec((1, S, grouped.shape[2]), lambda b: (b, 0, 0)),
        pl.BlockSpec((1, S, l2_pts_t.shape[2]), lambda b: (b, 0, 0)),
    ]
    for wb in wbs:
        in_specs.append(pl.BlockSpec(wb.shape, lambda b: (0, 0)))
    l3, l2p = pl.pallas_call(
        _sa3_fp3_body,
        grid=(B,),
        in_specs=in_specs[:1] + in_specs[1:],
        out_specs=[
            pl.BlockSpec((1, 1, 1024), lambda b: (b, 0, 0)),
            pl.BlockSpec((1, S, 256), lambda b: (b, 0, 0)),
        ],
        out_shape=[
            jax.ShapeDtypeStruct((B, 1, 1024), jnp.float32),
            jax.ShapeDtypeStruct((B, S, 256), jnp.float32),
        ],
    )(grouped, l2_pts_t, *wbs)
    return l3[:, 0, :], l2p


def _head_body(x_ref, w1_ref, b1_ref, w2_ref, b2_ref, o_ref):
    x = x_ref[0]
    h = jnp.dot(w1_ref[...], x, preferred_element_type=jnp.float32)
    h = h + b1_ref[...][:, :1]
    h = jax.nn.relu(h)
    h = jnp.dot(w2_ref[...], h, preferred_element_type=jnp.float32)
    h = h + b2_ref[...][:, :1]
    o_ref[0] = jax.nn.sigmoid(h)


def _head_pallas(l0p, head1, head2):
    B, C, N = l0p.shape
    BLK = 1024
    w1 = head1['W'] * (BN_SCALE * head1['gamma'])[:, None]
    b1 = (head1['b'] * BN_SCALE * head1['gamma'] + head1['beta'])[:, None]
    w2 = head2['W']
    b2 = head2['b'][:, None]
    out = pl.pallas_call(
        _head_body,
        grid=(B, N // BLK),
        in_specs=[
            pl.BlockSpec((1, C, BLK), lambda b, n: (b, 0, n)),
            pl.BlockSpec((w1.shape[0], C), lambda b, n: (0, 0)),
            pl.BlockSpec((w1.shape[0], 1), lambda b, n: (0, 0)),
            pl.BlockSpec((8, w2.shape[1]), lambda b, n: (0, 0)),
            pl.BlockSpec((8, 1), lambda b, n: (0, 0)),
        ],
        out_specs=pl.BlockSpec((1, 8, BLK), lambda b, n: (b, 0, n)),
        out_shape=jax.ShapeDtypeStruct((B, 8, N), jnp.float32),
    )(l0p, w1, b1, jnp.pad(w2, ((0, 2), (0, 0))), jnp.pad(b2, ((0, 2), (0, 0))))
    return out[:, :NUM_CLASSES, :]


def kernel(point_cloud_xyz, params):
    input_points = jnp.transpose(point_cloud_xyz, (0, 2, 1))
    input_xyz = input_points
    l1_xyz, l1_points = _sa_msg(input_xyz, input_points, 512, [0.1, 0.2, 0.4], [32, 64, 128], params['sa1'])
    l2_xyz, l2_points = _sa_msg(l1_xyz, l1_points, 128, [0.4, 0.8], [64, 128], params['sa2'])
    l2_xyz_t = jnp.transpose(l2_xyz, (0, 2, 1))
    l2_pts_t = jnp.transpose(l2_points, (0, 2, 1))
    l3_points, l2p_t = _sa3_fp3_pallas(l2_xyz_t, l2_pts_t, params['sa3'], params['fp3'])
    l1_xyz_t = jnp.transpose(l1_xyz, (0, 2, 1))
    l1_pts_t = jnp.transpose(l1_points, (0, 2, 1))
    l1p_t = _fp_pallas(l1_xyz_t, l2_xyz_t, l1_pts_t, l2p_t, params['fp2'])
    out = _fp_pallas(point_cloud_xyz, l1_xyz_t, point_cloud_xyz, l1p_t,
                     params['fp1'], heads=(params['head1'], params['head2']),
                     nblk=2048)
    return out[:, :, :NUM_CLASSES], l3_points[:, :, None]


# speedup vs baseline: 23.4663x; 8.7020x over previous
import functools

import jax
import jax.numpy as jnp
import numpy as np
from jax.experimental import pallas as pl
from jax.experimental.pallas import tpu as pltpu

NUM_CLASSES = 6
BN_EPS = 1e-5
BN_SCALE = 1.0 / np.sqrt(1.0 + BN_EPS)


def _square_distance(src, dst):
    return (
        jnp.sum(src ** 2, -1)[:, :, None]
        + jnp.sum(dst ** 2, -1)[:, None, :]
        - 2.0 * jnp.einsum('bnc,bmc->bnm', src, dst)
    )


def _index_points(points, idx):
    return jax.vmap(lambda p, i: p[i])(points, idx)


def _fps_body(npoint, x_ref, o_ref):
    B = x_ref.shape[0] // 3
    N = x_ref.shape[1]
    x = x_ref[0:B, :]
    y = x_ref[B:2 * B, :]
    z = x_ref[2 * B:3 * B, :]
    lane = jax.lax.broadcasted_iota(jnp.int32, (B, N), 1).astype(jnp.float32)
    lane_s = jax.lax.broadcasted_iota(jnp.int32, (B, npoint), 1)

    def step(s, state):
        distance, far, ax, ay, az = state
        oh = (lane == far).astype(jnp.float32)
        cx = jnp.sum(x * oh, axis=1, keepdims=True)
        cy = jnp.sum(y * oh, axis=1, keepdims=True)
        cz = jnp.sum(z * oh, axis=1, keepdims=True)
        ohs = (lane_s == s).astype(jnp.float32)
        ax = ax + cx * ohs
        ay = ay + cy * ohs
        az = az + cz * ohs
        dx = x - cx
        dy = y - cy
        dz = z - cz
        dist = dx * dx + dy * dy + dz * dz
        distance = jnp.minimum(distance, dist)
        m = jnp.max(distance, axis=1, keepdims=True)
        far = jnp.min(jnp.where(distance == m, lane, float(N)), axis=1, keepdims=True)
        return distance, far, ax, ay, az

    init = (
        jnp.full((B, N), 1e10, jnp.float32),
        jnp.zeros((B, 1), jnp.float32),
        jnp.zeros((B, npoint), jnp.float32),
        jnp.zeros((B, npoint), jnp.float32),
        jnp.zeros((B, npoint), jnp.float32),
    )
    _, _, ax, ay, az = jax.lax.fori_loop(0, npoint, step, init)
    o_ref[0:B, :] = ax
    o_ref[B:2 * B, :] = ay
    o_ref[2 * B:3 * B, :] = az


def _fps_new_xyz(xyz_t, npoint):
    B, N, _ = xyz_t.shape
    planes = jnp.transpose(xyz_t, (2, 0, 1)).reshape(3 * B, N)
    out = pl.pallas_call(
        functools.partial(_fps_body, npoint),
        in_specs=[pl.BlockSpec((3 * B, N), lambda: (0, 0))],
        out_specs=pl.BlockSpec((3 * B, npoint), lambda: (0, 0)),
        out_shape=jax.ShapeDtypeStruct((3 * B, npoint), jnp.float32),
    )(planes)
    return jnp.transpose(out.reshape(3, B, npoint), (1, 2, 0))


def _ballq_body(radii, ksamples, q_ref, x3_ref, *o_refs):
    q = q_ref[0]
    x3 = x3_ref[0]
    SB = q.shape[0]
    N = x3.shape[1]
    C = 128
    NC = N // C
    qn = jnp.sum(q * q, axis=1, keepdims=True)
    xn = jnp.sum(x3 * x3, axis=0, keepdims=True)
    qx = jax.lax.dot_general(q, x3, (((1,), (0,)), ((), ())),
                             preferred_element_type=jnp.float32)
    sqd = qn + xn - 2.0 * qx

    ii = jax.lax.broadcasted_iota(jnp.int32, (C, C), 0)
    jj = jax.lax.broadcasted_iota(jnp.int32, (C, C), 1)
    t128 = (ii <= jj).astype(jnp.bfloat16)
    i64 = jax.lax.broadcasted_iota(jnp.int32, (NC, NC), 0)
    j64 = jax.lax.broadcasted_iota(jnp.int32, (NC, NC), 1)
    t64 = (i64 <= j64).astype(jnp.bfloat16)
    c_iota = jax.lax.broadcasted_iota(jnp.int32, (1, NC, 1), 1).astype(jnp.float32)

    for i, (radius, K) in enumerate(zip(radii, ksamples)):
        kio = jax.lax.broadcasted_iota(jnp.int32, (SB, K), 1).astype(jnp.float32)
        mask = (sqd <= radius * radius).astype(jnp.bfloat16)
        rloc = jnp.dot(mask.reshape(SB * NC, C), t128,
                       preferred_element_type=jnp.float32)
        rloc3 = rloc.reshape(SB, NC, C)
        cnt = rloc3[:, :, C - 1]
        y = jnp.dot(cnt.astype(jnp.bfloat16), t64,
                    preferred_element_type=jnp.float32)
        xp = y - cnt
        ch = jnp.sum((y[:, :, None] <= kio[:, None, :]).astype(jnp.float32),
                     axis=1)
        chc = jnp.minimum(ch, float(NC - 1))
        oh = (chc[:, None, :] == c_iota).astype(jnp.float32)
        base = jnp.sum(oh * xp[:, :, None], axis=1)
        g = jax.lax.dot_general(
            oh.astype(jnp.bfloat16), rloc3.astype(jnp.bfloat16),
            (((1,), (1,)), ((0,), (0,))),
            preferred_element_type=jnp.float32)
        t = kio - base + 1.0
        lpos = jnp.sum((g < t[:, :, None]).astype(jnp.float32), axis=2)
        idx = chc * float(C) + lpos
        total = y[:, NC - 1][:, None]
        idx = jnp.where(kio < total, idx, idx[:, 0:1])
        o_refs[i][0] = idx.astype(jnp.int32)


def _query_ball_multi(radii, ksamples, xyz_t, new_xyz):
    B, N, _ = xyz_t.shape
    S = new_xyz.shape[1]
    SB = min(S, 128)
    x3 = jnp.transpose(xyz_t, (0, 2, 1))
    outs = pl.pallas_call(
        functools.partial(_ballq_body, radii, ksamples),
        grid=(B, S // SB),
        in_specs=[
            pl.BlockSpec((1, SB, 3), lambda b, s: (b, s, 0)),
            pl.BlockSpec((1, 3, N), lambda b, s: (b, 0, 0)),
        ],
        out_specs=[pl.BlockSpec((1, SB, K), lambda b, s: (b, s, 0))
                   for K in ksamples],
        out_shape=[jax.ShapeDtypeStruct((B, S, K), jnp.int32)
                   for K in ksamples],
    )(new_xyz, x3)
    return outs


def _sc_gather(table, idx, chunk):
    from jax.experimental.pallas import tpu_sc as plsc

    V, D = table.shape
    R = idx.shape[0]
    info = plsc.get_sparse_core_info()
    nw = info.num_cores * info.num_subcores
    b_per_w = R // nw
    ch = min(chunk, b_per_w)
    n_iter = b_per_w // ch
    mesh = plsc.VectorSubcoreMesh(core_axis_name="c", subcore_axis_name="s")

    @functools.partial(
        pl.kernel, mesh=mesh,
        out_type=jax.ShapeDtypeStruct((R, D), jnp.float32),
        scratch_types=[
            pltpu.VMEM((ch,), jnp.int32),
            pltpu.VMEM((ch, D), jnp.float32),
            pltpu.SemaphoreType.DMA,
        ],
    )
    def k(table_hbm, idx_hbm, out_hbm, idx_v, rows_v, sem):
        wid = jax.lax.axis_index("s") * info.num_cores + jax.lax.axis_index("c")
        base = wid * b_per_w
        for it in range(n_iter):
            off = base + it * ch
            pltpu.sync_copy(idx_hbm.at[pl.ds(off, ch)], idx_v)
            pltpu.async_copy(table_hbm.at[idx_v], rows_v, sem).wait()
            pltpu.sync_copy(rows_v, out_hbm.at[pl.ds(off, ch)])

    return k(table, idx)


def _sa_mlp_body(K, has_first, xg_ref, q_ref, *refs):
    w_refs = refs[:-1]
    o_ref = refs[-1]
    i = 0
    a_ref = None
    if has_first:
        a_ref = w_refs[0]
        i = 1
    qw_ref, qb_ref = w_refs[i], w_refs[i + 1]
    layer_refs = w_refs[i + 2:]
    xg = xg_ref[0]
    q = q_ref[0]
    SB = q.shape[0]
    qt = jnp.dot(q, qw_ref[...], preferred_element_type=jnp.float32) + qb_ref[...]
    h = xg
    if has_first:
        h = jnp.dot(h, a_ref[...], preferred_element_type=jnp.float32)
    c1 = h.shape[1]
    h = h.reshape(SB, K, c1) + qt[:, None, :]
    h = jax.nn.relu(h).reshape(SB * K, c1)
    for li in range(len(layer_refs) // 2):
        w, b = layer_refs[2 * li][...], layer_refs[2 * li + 1][...]
        h = jax.nn.relu(jnp.dot(h, w, preferred_element_type=jnp.float32) + b)
    co = h.shape[1]
    o_ref[0] = jnp.max(h.reshape(SB, K, co), axis=1)


def _sa_branch_mlp(xg, q8, a_mat, qw, qb, rest_wbs, K):
    B, SK, D = xg.shape
    S = SK // K
    SB = max(1, min(S, 8192 // K))
    wbs = ([a_mat] if a_mat is not None else []) + [qw, qb] + rest_wbs
    cout = rest_wbs[-2].shape[1] if rest_wbs else qw.shape[1]
    in_specs = [
        pl.BlockSpec((1, SB * K, D), lambda b, s: (b, s, 0)),
        pl.BlockSpec((1, SB, 8), lambda b, s: (b, s, 0)),
    ]
    for wb in wbs:
        in_specs.append(pl.BlockSpec(wb.shape, lambda b, s: (0, 0)))
    return pl.pallas_call(
        functools.partial(_sa_mlp_body, K, a_mat is not None),
        grid=(B, S // SB),
        in_specs=in_specs,
        out_specs=pl.BlockSpec((1, SB, cout), lambda b, s: (b, s, 0)),
        out_shape=jax.ShapeDtypeStruct((B, S, cout), jnp.float32),
    )(xg, q8, *wbs)


def _table_matmul_body(x_ref, w_ref, o_ref):
    o_ref[0] = jnp.dot(x_ref[0], w_ref[...], preferred_element_type=jnp.float32)


def _table_matmul(x, w):
    B, N, C = x.shape
    H = w.shape[1]
    return pl.pallas_call(
        _table_matmul_body,
        grid=(B,),
        in_specs=[
            pl.BlockSpec((1, N, C), lambda b: (b, 0, 0)),
            pl.BlockSpec((C, H), lambda b: (0, 0)),
        ],
        out_specs=pl.BlockSpec((1, N, H), lambda b: (b, 0, 0)),
        out_shape=jax.ShapeDtypeStruct((B, N, H), jnp.float32),
    )(x, w)


def _mlp2d(x, layers):
    for p in layers:
        x = jnp.einsum('oc,bcns->bons', p['W'], x) + p['b'][None, :, None, None]
        x = x * BN_SCALE * p['gamma'][None, :, None, None] + p['beta'][None, :, None, None]
        x = jax.nn.relu(x)
    return x


def _mlp1d(x, layers):
    for p in layers:
        x = jnp.einsum('oc,bcn->bon', p['W'], x) + p['b'][None, :, None]
        x = x * BN_SCALE * p['gamma'][None, :, None] + p['beta'][None, :, None]
        x = jax.nn.relu(x)
    return x


def _sa_msg_pl(xyz_t, pts_t, npoint, radii, nsamples, branches):
    B, N, _ = xyz_t.shape
    new_xyz = _fps_new_xyz(xyz_t, npoint)
    gidxs = _query_ball_multi(radii, nsamples, xyz_t, new_xyz)
    q8 = jnp.pad(new_xyz, ((0, 0), (0, 0), (0, 5)))
    boff = (jnp.arange(B, dtype=jnp.int32) * N)[:, None, None]
    if pts_t is None:
        feats_in = jnp.pad(xyz_t, ((0, 0), (0, 0), (0, 5)))
    else:
        feats_in = jnp.concatenate([pts_t, xyz_t], axis=2)
    outs = []
    for gidx, K, layers in zip(gidxs, nsamples, branches):
        l0 = layers[0]
        s0 = BN_SCALE * l0['gamma']
        wfull = l0['W'] * s0[:, None]
        bfull = l0['b'] * s0 + l0['beta']
        c1 = wfull.shape[0]
        cin = wfull.shape[1]
        wx = wfull[:, cin - 3:]
        if pts_t is None:
            wt_point = jnp.pad(wfull[:, :3] + wx, ((0, 0), (0, 5))).T
        else:
            wt_point = wfull.T
        wt_point = jnp.pad(wt_point, ((0, 0), (0, 128 - c1)))
        table = _table_matmul(feats_in, wt_point).reshape(B * N, 128)
        qw = jnp.pad(-wx.T, ((0, 5), (0, 128 - c1)))
        qb = jnp.pad(bfull[None, :], ((0, 0), (0, 128 - c1)))
        rest = []
        for i, p in enumerate(layers[1:]):
            wt, bb = _fuse_bn(p)
            if i == 0:
                wt = jnp.pad(wt, ((0, 128 - c1), (0, 0)))
            rest += [wt, bb]
        idxg = (gidx + boff).reshape(-1)
        xg = _sc_gather(table, idxg, 512)
        xg = xg.reshape(B, npoint * K, 128)
        outs.append(_sa_branch_mlp(xg, q8, None, qw, qb, rest, K))
    return new_xyz, jnp.concatenate(outs, axis=2)


def _sa_group_all(xyz, points, layers):
    xyz_t = jnp.transpose(xyz, (0, 2, 1))
    pts_t = jnp.transpose(points, (0, 2, 1))
    B = xyz_t.shape[0]
    grouped = jnp.concatenate([xyz_t, pts_t], axis=-1)[:, None, :, :]
    h = jnp.transpose(grouped, (0, 3, 2, 1))
    h = _mlp2d(h, layers)
    new_points = jnp.max(h, axis=2)
    return jnp.zeros((B, 3, 1), xyz.dtype), new_points


def _feature_propagation(xyz1, xyz2, points1, points2, layers):
    xyz1_t = jnp.transpose(xyz1, (0, 2, 1))
    xyz2_t = jnp.transpose(xyz2, (0, 2, 1))
    pts2_t = jnp.transpose(points2, (0, 2, 1))
    N = xyz1_t.shape[1]
    S = xyz2_t.shape[1]
    if S == 1:
        interpolated = jnp.repeat(pts2_t, N, axis=1)
    else:
        dists = _square_distance(xyz1_t, xyz2_t)
        idx = jnp.argsort(dists, axis=-1)[:, :, :3]
        d = jnp.take_along_axis(dists, idx, axis=-1)
        recip = 1.0 / (d + 1e-8)
        weight = recip / jnp.sum(recip, axis=2, keepdims=True)
        interpolated = jnp.sum(_index_points(pts2_t, idx) * weight[..., None], axis=2)
    new_points = jnp.concatenate([jnp.transpose(points1, (0, 2, 1)), interpolated], axis=-1)
    return _mlp1d(jnp.transpose(new_points, (0, 2, 1)), layers)


def _fuse_bn(p):
    s = BN_SCALE * p['gamma']
    wt = (p['W'] * s[:, None]).T
    b = (p['b'] * s + p['beta'])[None, :]
    return wt, b


def _fp_body(nlayers, nheads, q_ref, x3_ref, p1_ref, p2_ref, *refs):
    w_refs = refs[:-1]
    o_ref = refs[-1]
    q = q_ref[0]
    x3 = x3_ref[0]
    S = x3.shape[1]
    NB = q.shape[0]
    qn = jnp.sum(q * q, axis=1, keepdims=True)
    xn = jnp.sum(x3 * x3, axis=0, keepdims=True)
    qx = jax.lax.dot_general(q, x3, (((1,), (0,)), ((), ())),
                             preferred_element_type=jnp.float32)
    sqd = qn + xn - 2.0 * qx
    lane = jax.lax.broadcasted_iota(jnp.int32, (NB, S), 1).astype(jnp.float32)
    p2 = p2_ref[0]
    interp = None
    rsum = None
    dists = sqd
    parts = []
    for _ in range(3):
        m = jnp.min(dists, axis=1, keepdims=True)
        pos = jnp.min(jnp.where(dists == m, lane, float(S)), axis=1, keepdims=True)
        oh = (lane == pos).astype(jnp.float32)
        gath = jnp.dot(oh, p2, preferred_element_type=jnp.float32)
        r = 1.0 / (m + 1e-8)
        parts.append((r, gath))
        rsum = r if rsum is None else rsum + r
        dists = jnp.where(oh > 0, jnp.float32(3.4e38), dists)
    interp = sum((r / rsum) * g for r, g in parts)
    h = jnp.concatenate([p1_ref[0], interp], axis=1)
    for li in range(nlayers):
        w, b = w_refs[2 * li][...], w_refs[2 * li + 1][...]
        h = jnp.dot(h, w, preferred_element_type=jnp.float32) + b
        h = jax.nn.relu(h)
    if nheads:
        w, b = w_refs[2 * nlayers][...], w_refs[2 * nlayers + 1][...]
        h = jax.nn.relu(jnp.dot(h, w, preferred_element_type=jnp.float32) + b)
        w, b = w_refs[2 * nlayers + 2][...], w_refs[2 * nlayers + 3][...]
        h = jax.nn.sigmoid(jnp.dot(h, w, preferred_element_type=jnp.float32) + b)
    o_ref[0] = h


def _fp_pallas(xyz1_t, xyz2_t, pts1_t, pts2_t, layers, heads=None, nblk=None):
    B, N, _ = xyz1_t.shape
    S = xyz2_t.shape[1]
    C1 = pts1_t.shape[2]
    C2 = pts2_t.shape[2]
    NB = nblk or N
    x3 = jnp.transpose(xyz2_t, (0, 2, 1))
    wbs = []
    for p in layers:
        wt, b = _fuse_bn(p)
        wbs += [wt, b]
    nheads = 0
    if heads is not None:
        h1, h2 = heads
        wt, b = _fuse_bn(h1)
        wbs += [wt, b]
        w2 = jnp.pad(h2['W'].T, ((0, 0), (0, 8 - NUM_CLASSES)))
        b2 = jnp.pad(h2['b'][None, :], ((0, 0), (0, 8 - NUM_CLASSES)))
        wbs += [w2, b2]
        nheads = 2
    cout = wbs[-2].shape[1]
    in_specs = [
        pl.BlockSpec((1, NB, 3), lambda b, n: (b, n, 0)),
        pl.BlockSpec((1, 3, S), lambda b, n: (b, 0, 0)),
        pl.BlockSpec((1, NB, C1), lambda b, n: (b, n, 0)),
        pl.BlockSpec((1, S, C2), lambda b, n: (b, 0, 0)),
    ]
    for wb in wbs:
        in_specs.append(pl.BlockSpec(wb.shape, lambda b, n: tuple([0] * wb.ndim)))
    out = pl.pallas_call(
        functools.partial(_fp_body, len(layers), nheads),
        grid=(B, N // NB),
        in_specs=in_specs,
        out_specs=pl.BlockSpec((1, NB, cout), lambda b, n: (b, n, 0)),
        out_shape=jax.ShapeDtypeStruct((B, N, cout), jnp.float32),
    )(xyz1_t, x3, pts1_t, pts2_t, *wbs)
    return out


def _sa3_fp3_body(p1_ref, px_ref, *refs):
    w_refs = refs[:-2]
    l3_ref, o_ref = refs[-2], refs[-1]
    h = p1_ref[0]
    for li in range(3):
        w, b = w_refs[2 * li][...], w_refs[2 * li + 1][...]
        h = jax.nn.relu(jnp.dot(h, w, preferred_element_type=jnp.float32) + b)
    l3 = jnp.max(h, axis=0, keepdims=True)
    l3_ref[0] = l3
    S = px_ref.shape[1]
    g = jnp.concatenate(
        [px_ref[0], jnp.broadcast_to(l3, (S, l3.shape[1]))], axis=1)
    for li in range(3, 5):
        w, b = w_refs[2 * li][...], w_refs[2 * li + 1][...]
        g = jax.nn.relu(jnp.dot(g, w, preferred_element_type=jnp.float32) + b)
    o_ref[0] = g


def _sa3_fp3_pallas(l2_xyz_t, l2_pts_t, sa3_layers, fp3_layers):
    B, S, _ = l2_xyz_t.shape
    grouped = jnp.concatenate([l2_xyz_t, l2_pts_t], axis=2)
    wbs = []
    for p in list(sa3_layers) + list(fp3_layers):
        wt, b = _fuse_bn(p)
        wbs += [wt, b]
    in_specs = [
        pl.BlockSpec((1, S, grouped.shape[2]), lambda b: (b, 0, 0)),
        pl.BlockSpec((1, S, l2_pts_t.shape[2]), lambda b: (b, 0, 0)),
    ]
    for wb in wbs:
        in_specs.append(pl.BlockSpec(wb.shape, lambda b: (0, 0)))
    l3, l2p = pl.pallas_call(
        _sa3_fp3_body,
        grid=(B,),
        in_specs=in_specs[:1] + in_specs[1:],
        out_specs=[
            pl.BlockSpec((1, 1, 1024), lambda b: (b, 0, 0)),
            pl.BlockSpec((1, S, 256), lambda b: (b, 0, 0)),
        ],
        out_shape=[
            jax.ShapeDtypeStruct((B, 1, 1024), jnp.float32),
            jax.ShapeDtypeStruct((B, S, 256), jnp.float32),
        ],
    )(grouped, l2_pts_t, *wbs)
    return l3[:, 0, :], l2p


def _head_body(x_ref, w1_ref, b1_ref, w2_ref, b2_ref, o_ref):
    x = x_ref[0]
    h = jnp.dot(w1_ref[...], x, preferred_element_type=jnp.float32)
    h = h + b1_ref[...][:, :1]
    h = jax.nn.relu(h)
    h = jnp.dot(w2_ref[...], h, preferred_element_type=jnp.float32)
    h = h + b2_ref[...][:, :1]
    o_ref[0] = jax.nn.sigmoid(h)


def _head_pallas(l0p, head1, head2):
    B, C, N = l0p.shape
    BLK = 1024
    w1 = head1['W'] * (BN_SCALE * head1['gamma'])[:, None]
    b1 = (head1['b'] * BN_SCALE * head1['gamma'] + head1['beta'])[:, None]
    w2 = head2['W']
    b2 = head2['b'][:, None]
    out = pl.pallas_call(
        _head_body,
        grid=(B, N // BLK),
        in_specs=[
            pl.BlockSpec((1, C, BLK), lambda b, n: (b, 0, n)),
            pl.BlockSpec((w1.shape[0], C), lambda b, n: (0, 0)),
            pl.BlockSpec((w1.shape[0], 1), lambda b, n: (0, 0)),
            pl.BlockSpec((8, w2.shape[1]), lambda b, n: (0, 0)),
            pl.BlockSpec((8, 1), lambda b, n: (0, 0)),
        ],
        out_specs=pl.BlockSpec((1, 8, BLK), lambda b, n: (b, 0, n)),
        out_shape=jax.ShapeDtypeStruct((B, 8, N), jnp.float32),
    )(l0p, w1, b1, jnp.pad(w2, ((0, 2), (0, 0))), jnp.pad(b2, ((0, 2), (0, 0))))
    return out[:, :NUM_CLASSES, :]


def kernel(point_cloud_xyz, params):
    nx1, l1f = _sa_msg_pl(point_cloud_xyz, None, 512,
                          [0.1, 0.2, 0.4], [32, 64, 128], params['sa1'])
    nx2, l2f = _sa_msg_pl(nx1, l1f, 128, [0.4, 0.8], [64, 128], params['sa2'])
    l3_points, l2p_t = _sa3_fp3_pallas(nx2, l2f, params['sa3'], params['fp3'])
    l1p_t = _fp_pallas(nx1, nx2, l1f, l2p_t, params['fp2'])
    out = _fp_pallas(point_cloud_xyz, nx1, point_cloud_xyz, l1p_t,
                     params['fp1'], heads=(params['head1'], params['head2']),
                     nblk=2048)
    return out[:, :, :NUM_CLASSES], l3_points[:, :, None]
